# baseline (speedup 1.0000x reference)
import functools

import jax
import jax.numpy as jnp
from jax import lax
from jax.experimental import pallas as pl
from jax.experimental.pallas import tpu as pltpu
from jax.experimental.pallas import tpu_sc as plsc

VOCAB = 1000000
EMBED = 64
B = 16384
P = 20
N = 100

NC = 2
NS = 16
L = 16
NW = NC * NS
BW = B // NW
BC = 8
NCHUNK = BW // BC
CTX = P + N
NG = 8


def _sc_dots():
    mesh = plsc.VectorSubcoreMesh(core_axis_name="c", subcore_axis_name="s")

    @functools.partial(
        pl.kernel,
        mesh=mesh,
        out_type=jax.ShapeDtypeStruct((B, 128), jnp.float32),
        compiler_params=pltpu.CompilerParams(
            needs_layout_passes=False, use_tc_tiling_on_sc=False),
        scratch_types=[
            pltpu.VMEM((BC,), jnp.int32),
            pltpu.VMEM((BC,), jnp.int32),
            pltpu.VMEM((BC, P), jnp.int32),
            pltpu.VMEM((BC, P), jnp.int32),
            pltpu.VMEM((BC, N), jnp.int32),
            pltpu.VMEM((BC, N), jnp.int32),
            pltpu.VMEM((BC, EMBED), jnp.float32),
            pltpu.VMEM((BC, EMBED), jnp.float32),
            pltpu.VMEM((BC * CTX + 8, EMBED), jnp.float32),
            pltpu.VMEM((BC * CTX + 8, EMBED), jnp.float32),
            pltpu.VMEM((BC, 128), jnp.float32),
            pltpu.VMEM((BC, 128), jnp.float32),
            pltpu.SemaphoreType.DMA,
            pltpu.SemaphoreType.DMA,
            pltpu.SemaphoreType.DMA,
            pltpu.SemaphoreType.DMA,
            pltpu.SemaphoreType.DMA,
            pltpu.SemaphoreType.DMA,
        ],
    )
    def k(input_hbm, pos_hbm, neg_hbm, ine_hbm, oute_hbm, dots_hbm,
          inb0, inb1, pb0, pb1, nb0, nb1, ub0, ub1, rw0, rw1, os0, os1,
          sg0, sg1, so0, so1, si0, si1):
        inb, pb, nb, ub, rw, osb = (inb0, inb1), (pb0, pb1), (nb0, nb1), \
            (ub0, ub1), (rw0, rw1), (os0, os1)
        sg, so, si = (sg0, sg1), (so0, so1), (si0, si1)
        wid = lax.axis_index("s") * NC + lax.axis_index("c")
        lane = lax.iota(jnp.int32, L)
        lane16 = lane * L

        def stage_idx(ci, kb):
            b0 = wid * BW + ci * BC
            pltpu.async_copy(input_hbm.at[pl.ds(b0, BC)], inb[kb], si[kb])
            pltpu.async_copy(pos_hbm.at[pl.ds(b0, BC)], pb[kb], si[kb])
            pltpu.async_copy(neg_hbm.at[pl.ds(b0, BC)], nb[kb], si[kb])

        def wait_idx(kb):
            pltpu.make_async_copy(
                input_hbm.at[pl.ds(0, BC)], inb[kb], si[kb]).wait()
            pltpu.make_async_copy(
                pos_hbm.at[pl.ds(0, BC)], pb[kb], si[kb]).wait()
            pltpu.make_async_copy(
                neg_hbm.at[pl.ds(0, BC)], nb[kb], si[kb]).wait()

        def fire_gathers(kb):
            pltpu.async_copy(ine_hbm.at[inb[kb]], ub[kb], sg[kb])
            for b in range(BC):
                pltpu.async_copy(oute_hbm.at[pb[kb].at[b]],
                                 rw[kb].at[pl.ds(b * CTX, P)], sg[kb])
                pltpu.async_copy(oute_hbm.at[nb[kb].at[b]],
                                 rw[kb].at[pl.ds(b * CTX + P, N)], sg[kb])

        def wait_gathers(kb):
            pltpu.make_async_copy(
                ine_hbm.at[pl.ds(0, BC)], ub[kb], sg[kb]).wait()
            pltpu.make_async_copy(
                oute_hbm.at[pl.ds(0, BC * CTX)],
                rw[kb].at[pl.ds(0, BC * CTX)], sg[kb]).wait()

        def compute(kb):
            rows, ubuf, ostage = rw[kb], ub[kb], osb[kb]

            def b_body(b, _):
                bvec = jnp.full((L,), 0, jnp.int32) + b

                def g_body(g, _):
                    rvec = (b * CTX + g * L) + lane
                    zero = jnp.zeros((L,), jnp.float32)

                    def d_body(i, accs):
                        out = []
                        for j in range(4):
                            dvec = jnp.full((L,), 0, jnp.int32) + (i * 4 + j)
                            col = plsc.load_gather(rows, [rvec, dvec])
                            usd = plsc.load_gather(ubuf, [bvec, dvec])
                            out.append(accs[j] + col * usd)
                        return tuple(out)

                    accs = lax.fori_loop(
                        0, EMBED // 4, d_body, (zero, zero, zero, zero))
                    dot = (accs[0] + accs[1]) + (accs[2] + accs[3])
                    ostage[b, pl.ds(g * L, L)] = dot
                    return 0

                lax.fori_loop(0, NG, g_body, 0)
                return 0

            lax.fori_loop(0, BC, b_body, 0)

        stage_idx(0, 0)
        stage_idx(1, 1)
        wait_idx(0)
        fire_gathers(0)

        def loop_body(i, _):
            for kb in range(2):
                ci = 2 * i + kb

                @pl.when(ci + 1 < NCHUNK)
                def _():
                    wait_idx(1 - kb)
                    fire_gathers(1 - kb)

                wait_gathers(kb)

                @pl.when(ci + 2 < NCHUNK)
                def _():
                    stage_idx(ci + 2, kb)

                @pl.when(ci >= 2)
                def _():
                    b0_old = wid * BW + (ci - 2) * BC
                    pltpu.make_async_copy(
                        osb[kb], dots_hbm.at[pl.ds(b0_old, BC)],
                        so[kb]).wait()

                compute(kb)
                b0 = wid * BW + ci * BC
                pltpu.async_copy(osb[kb], dots_hbm.at[pl.ds(b0, BC)], so[kb])
            return 0

        lax.fori_loop(0, NCHUNK // 2, loop_body, 0)
        for kb in range(2):
            b0_last = wid * BW + (NCHUNK - 2 + kb) * BC
            pltpu.make_async_copy(
                osb[kb], dots_hbm.at[pl.ds(b0_last, BC)], so[kb]).wait()

    return k


def _tc_loss(dots):
    def body(d_ref, o_ref):
        x = d_ref[...]
        col = lax.broadcasted_iota(jnp.int32, x.shape, 1)
        s = jnp.where(col < P, x, -x)
        ls = jnp.where(col < CTX, jax.nn.log_sigmoid(s), 0.0)
        o_ref[...] = -jnp.sum(ls, axis=1)

    blk = 512
    return pl.pallas_call(
        body,
        grid=(B // blk,),
        in_specs=[pl.BlockSpec((blk, 128), lambda i: (i, 0))],
        out_specs=pl.BlockSpec((blk,), lambda i: (i,)),
        out_shape=jax.ShapeDtypeStruct((B,), jnp.float32),
    )(dots)


def kernel(input_labels, pos_labels, neg_labels, in_embed, out_embed):
    dots = _sc_dots()(
        input_labels.astype(jnp.int32),
        pos_labels.astype(jnp.int32),
        neg_labels.astype(jnp.int32),
        in_embed,
        out_embed,
    )
    return _tc_loss(dots)


# speedup vs baseline: 1.1578x; 1.1578x over previous
import functools

import jax
import jax.numpy as jnp
from jax import lax
from jax.experimental import pallas as pl
from jax.experimental.pallas import tpu as pltpu
from jax.experimental.pallas import tpu_sc as plsc

VOCAB = 1000000
EMBED = 64
B = 16384
P = 20
N = 100

NC = 2
NS = 16
L = 16
NW = NC * NS
BW = B // NW
BC = 8
NCHUNK = BW // BC
CTX = P + N
NG = 8


def _sc_dots():
    mesh = plsc.VectorSubcoreMesh(core_axis_name="c", subcore_axis_name="s")

    @functools.partial(
        pl.kernel,
        mesh=mesh,
        out_type=jax.ShapeDtypeStruct((B, 128), jnp.float32),
        compiler_params=pltpu.CompilerParams(
            needs_layout_passes=False, use_tc_tiling_on_sc=False),
        scratch_types=[
            pltpu.VMEM((BC,), jnp.int32),
            pltpu.VMEM((BC,), jnp.int32),
            pltpu.VMEM((BC, P), jnp.int32),
            pltpu.VMEM((BC, P), jnp.int32),
            pltpu.VMEM((BC, N), jnp.int32),
            pltpu.VMEM((BC, N), jnp.int32),
            pltpu.VMEM((BC, EMBED), jnp.float32),
            pltpu.VMEM((BC, EMBED), jnp.float32),
            pltpu.VMEM((BC * CTX + 8, EMBED), jnp.float32),
            pltpu.VMEM((BC * CTX + 8, EMBED), jnp.float32),
            pltpu.VMEM((BC, 128), jnp.float32),
            pltpu.VMEM((BC, 128), jnp.float32),
            pltpu.SemaphoreType.DMA,
            pltpu.SemaphoreType.DMA,
            pltpu.SemaphoreType.DMA,
            pltpu.SemaphoreType.DMA,
            pltpu.SemaphoreType.DMA,
            pltpu.SemaphoreType.DMA,
        ],
    )
    def k(input_hbm, pos_hbm, neg_hbm, ine_hbm, oute_hbm, dots_hbm,
          inb0, inb1, pb0, pb1, nb0, nb1, ub0, ub1, rw0, rw1, os0, os1,
          sg0, sg1, so0, so1, si0, si1):
        inb, pb, nb, ub, rw, osb = (inb0, inb1), (pb0, pb1), (nb0, nb1), \
            (ub0, ub1), (rw0, rw1), (os0, os1)
        sg, so, si = (sg0, sg1), (so0, so1), (si0, si1)
        wid = lax.axis_index("s") * NC + lax.axis_index("c")
        lane = lax.iota(jnp.int32, L)
        lane16 = lane * L

        def stage_idx(ci, kb):
            b0 = wid * BW + ci * BC
            pltpu.async_copy(input_hbm.at[pl.ds(b0, BC)], inb[kb], si[kb])
            pltpu.async_copy(pos_hbm.at[pl.ds(b0, BC)], pb[kb], si[kb])
            pltpu.async_copy(neg_hbm.at[pl.ds(b0, BC)], nb[kb], si[kb])

        def wait_idx(kb):
            pltpu.make_async_copy(
                input_hbm.at[pl.ds(0, BC)], inb[kb], si[kb]).wait()
            pltpu.make_async_copy(
                pos_hbm.at[pl.ds(0, BC)], pb[kb], si[kb]).wait()
            pltpu.make_async_copy(
                neg_hbm.at[pl.ds(0, BC)], nb[kb], si[kb]).wait()

        def fire_gathers(kb):
            pltpu.async_copy(ine_hbm.at[inb[kb]], ub[kb], sg[kb])
            for b in range(BC):
                pltpu.async_copy(oute_hbm.at[pb[kb].at[b]],
                                 rw[kb].at[pl.ds(b * CTX, P)], sg[kb])
                pltpu.async_copy(oute_hbm.at[nb[kb].at[b]],
                                 rw[kb].at[pl.ds(b * CTX + P, N)], sg[kb])

        def wait_gathers(kb):
            pltpu.make_async_copy(
                ine_hbm.at[pl.ds(0, BC)], ub[kb], sg[kb]).wait()
            pltpu.make_async_copy(
                oute_hbm.at[pl.ds(0, BC * CTX)],
                rw[kb].at[pl.ds(0, BC * CTX)], sg[kb]).wait()

        zeros16 = jnp.zeros((L,), jnp.float32)

        def compute(kb):
            rows, ubuf, ostage = rw[kb], ub[kb], osb[kb]

            def b_body(b, _):
                u0 = ubuf[b, pl.ds(0, L)]
                u1 = ubuf[b, pl.ds(L, L)]
                u2 = ubuf[b, pl.ds(2 * L, L)]
                u3 = ubuf[b, pl.ds(3 * L, L)]
                bvec = jnp.full((L,), 0, jnp.int32) + b
                for j in range(NG):
                    ostage[b, pl.ds(j * L, L)] = zeros16

                def g_body(g, _):
                    r0 = b * CTX + g * L
                    cb = jnp.full((L,), 0, jnp.int32) + g * L
                    for c in range(L):
                        r = r0 + c
                        p = ((rows[r, pl.ds(0, L)] * u0
                              + rows[r, pl.ds(L, L)] * u1)
                             + (rows[r, pl.ds(2 * L, L)] * u2
                                + rows[r, pl.ds(3 * L, L)] * u3))
                        plsc.addupdate_scatter(ostage, [bvec, cb + c], p)
                    return 0

                lax.fori_loop(0, NG, g_body, 0)
                return 0

            lax.fori_loop(0, BC, b_body, 0)

        stage_idx(0, 0)
        stage_idx(1, 1)
        wait_idx(0)
        fire_gathers(0)

        def loop_body(i, _):
            for kb in range(2):
                ci = 2 * i + kb

                @pl.when(ci + 1 < NCHUNK)
                def _():
                    wait_idx(1 - kb)
                    fire_gathers(1 - kb)

                wait_gathers(kb)

                @pl.when(ci + 2 < NCHUNK)
                def _():
                    stage_idx(ci + 2, kb)

                @pl.when(ci >= 2)
                def _():
                    b0_old = wid * BW + (ci - 2) * BC
                    pltpu.make_async_copy(
                        osb[kb], dots_hbm.at[pl.ds(b0_old, BC)],
                        so[kb]).wait()

                compute(kb)
                b0 = wid * BW + ci * BC
                pltpu.async_copy(osb[kb], dots_hbm.at[pl.ds(b0, BC)], so[kb])
            return 0

        lax.fori_loop(0, NCHUNK // 2, loop_body, 0)
        for kb in range(2):
            b0_last = wid * BW + (NCHUNK - 2 + kb) * BC
            pltpu.make_async_copy(
                osb[kb], dots_hbm.at[pl.ds(b0_last, BC)], so[kb]).wait()

    return k


def _tc_loss(dots):
    def body(d_ref, o_ref):
        x = d_ref[...]
        col = lax.broadcasted_iota(jnp.int32, x.shape, 1)
        s = jnp.where(col < P, x, -x)
        ls = jnp.where(col < CTX, jax.nn.log_sigmoid(s), 0.0)
        o_ref[...] = -jnp.sum(ls, axis=1)

    blk = 512
    return pl.pallas_call(
        body,
        grid=(B // blk,),
        in_specs=[pl.BlockSpec((blk, 128), lambda i: (i, 0))],
        out_specs=pl.BlockSpec((blk,), lambda i: (i,)),
        out_shape=jax.ShapeDtypeStruct((B,), jnp.float32),
    )(dots)


def kernel(input_labels, pos_labels, neg_labels, in_embed, out_embed):
    dots = _sc_dots()(
        input_labels.astype(jnp.int32),
        pos_labels.astype(jnp.int32),
        neg_labels.astype(jnp.int32),
        in_embed,
        out_embed,
    )
    return _tc_loss(dots)


# speedup vs baseline: 1.7490x; 1.5106x over previous
import functools

import jax
import jax.numpy as jnp
from jax import lax
from jax.experimental import pallas as pl
from jax.experimental.pallas import tpu as pltpu
from jax.experimental.pallas import tpu_sc as plsc

VOCAB = 1000000
EMBED = 64
B = 16384
P = 20
N = 100

NC = 2
NS = 16
L = 16
NW = NC * NS
BW = B // NW
BC = 8
NCHUNK = BW // BC
CTX = P + N
NG = 8


def _sc_dots():
    mesh = plsc.VectorSubcoreMesh(core_axis_name="c", subcore_axis_name="s")

    @functools.partial(
        pl.kernel,
        mesh=mesh,
        out_type=jax.ShapeDtypeStruct((B, 128), jnp.float32),
        compiler_params=pltpu.CompilerParams(
            needs_layout_passes=False, use_tc_tiling_on_sc=False),
        scratch_types=[
            pltpu.VMEM((BC,), jnp.int32),
            pltpu.VMEM((BC,), jnp.int32),
            pltpu.VMEM((BC, P), jnp.int32),
            pltpu.VMEM((BC, P), jnp.int32),
            pltpu.VMEM((BC, N), jnp.int32),
            pltpu.VMEM((BC, N), jnp.int32),
            pltpu.VMEM((BC, EMBED), jnp.float32),
            pltpu.VMEM((BC, EMBED), jnp.float32),
            pltpu.VMEM((BC * CTX + 8, EMBED), jnp.float32),
            pltpu.VMEM((BC * CTX + 8, EMBED), jnp.float32),
            pltpu.VMEM((BC, 128), jnp.float32),
            pltpu.VMEM((BC, 128), jnp.float32),
            pltpu.VMEM((2 * L * L,), jnp.float32),
            pltpu.SemaphoreType.DMA,
            pltpu.SemaphoreType.DMA,
            pltpu.SemaphoreType.DMA,
            pltpu.SemaphoreType.DMA,
            pltpu.SemaphoreType.DMA,
            pltpu.SemaphoreType.DMA,
        ],
    )
    def k(input_hbm, pos_hbm, neg_hbm, ine_hbm, oute_hbm, dots_hbm,
          inb0, inb1, pb0, pb1, nb0, nb1, ub0, ub1, rw0, rw1, os0, os1, tbuf,
          sg0, sg1, so0, so1, si0, si1):
        inb, pb, nb, ub, rw, osb = (inb0, inb1), (pb0, pb1), (nb0, nb1), \
            (ub0, ub1), (rw0, rw1), (os0, os1)
        sg, so, si = (sg0, sg1), (so0, so1), (si0, si1)
        wid = lax.axis_index("s") * NC + lax.axis_index("c")
        lane = lax.iota(jnp.int32, L)
        lane16 = lane * L

        def stage_idx(ci, kb):
            b0 = wid * BW + ci * BC
            pltpu.async_copy(input_hbm.at[pl.ds(b0, BC)], inb[kb], si[kb])
            pltpu.async_copy(pos_hbm.at[pl.ds(b0, BC)], pb[kb], si[kb])
            pltpu.async_copy(neg_hbm.at[pl.ds(b0, BC)], nb[kb], si[kb])

        def wait_idx(kb):
            pltpu.make_async_copy(
                input_hbm.at[pl.ds(0, BC)], inb[kb], si[kb]).wait()
            pltpu.make_async_copy(
                pos_hbm.at[pl.ds(0, BC)], pb[kb], si[kb]).wait()
            pltpu.make_async_copy(
                neg_hbm.at[pl.ds(0, BC)], nb[kb], si[kb]).wait()

        def fire_gathers(kb):
            pltpu.async_copy(ine_hbm.at[inb[kb]], ub[kb], sg[kb])
            for b in range(BC):
                pltpu.async_copy(oute_hbm.at[pb[kb].at[b]],
                                 rw[kb].at[pl.ds(b * CTX, P)], sg[kb])
                pltpu.async_copy(oute_hbm.at[nb[kb].at[b]],
                                 rw[kb].at[pl.ds(b * CTX + P, N)], sg[kb])

        def wait_gathers(kb):
            pltpu.make_async_copy(
                ine_hbm.at[pl.ds(0, BC)], ub[kb], sg[kb]).wait()
            pltpu.make_async_copy(
                oute_hbm.at[pl.ds(0, BC * CTX)],
                rw[kb].at[pl.ds(0, BC * CTX)], sg[kb]).wait()

        def compute(kb):
            rows, ubuf, ostage = rw[kb], ub[kb], osb[kb]

            def b_body(b, _):
                u0 = ubuf[b, pl.ds(0, L)]
                u1 = ubuf[b, pl.ds(L, L)]
                u2 = ubuf[b, pl.ds(2 * L, L)]
                u3 = ubuf[b, pl.ds(3 * L, L)]

                def g_body(t, _):
                    for half in range(2):
                        g = 2 * t + half
                        r0 = b * CTX + g * L
                        tb = half * (L * L)
                        for c in range(L):
                            r = r0 + c
                            p = ((rows[r, pl.ds(0, L)] * u0
                                  + rows[r, pl.ds(L, L)] * u1)
                                 + (rows[r, pl.ds(2 * L, L)] * u2
                                    + rows[r, pl.ds(3 * L, L)] * u3))
                            tbuf[pl.ds(tb + c * L, L)] = p
                    for half in range(2):
                        g = 2 * t + half
                        base = lane16 + half * (L * L)
                        cs = [plsc.load_gather(tbuf, [base + d])
                              for d in range(L)]
                        while len(cs) > 1:
                            cs = [cs[i] + cs[i + 1]
                                  for i in range(0, len(cs), 2)]
                        ostage[b, pl.ds(g * L, L)] = cs[0]
                    return 0

                lax.fori_loop(0, NG // 2, g_body, 0)
                return 0

            lax.fori_loop(0, BC, b_body, 0)

        stage_idx(0, 0)
        stage_idx(1, 1)
        wait_idx(0)
        fire_gathers(0)

        def loop_body(i, _):
            for kb in range(2):
                ci = 2 * i + kb

                @pl.when(ci + 1 < NCHUNK)
                def _():
                    wait_idx(1 - kb)
                    fire_gathers(1 - kb)

                wait_gathers(kb)

                @pl.when(ci + 2 < NCHUNK)
                def _():
                    stage_idx(ci + 2, kb)

                @pl.when(ci >= 2)
                def _():
                    b0_old = wid * BW + (ci - 2) * BC
                    pltpu.make_async_copy(
                        osb[kb], dots_hbm.at[pl.ds(b0_old, BC)],
                        so[kb]).wait()

                compute(kb)
                b0 = wid * BW + ci * BC
                pltpu.async_copy(osb[kb], dots_hbm.at[pl.ds(b0, BC)], so[kb])
            return 0

        lax.fori_loop(0, NCHUNK // 2, loop_body, 0)
        for kb in range(2):
            b0_last = wid * BW + (NCHUNK - 2 + kb) * BC
            pltpu.make_async_copy(
                osb[kb], dots_hbm.at[pl.ds(b0_last, BC)], so[kb]).wait()

    return k


def _tc_loss(dots):
    def body(d_ref, o_ref):
        x = d_ref[...]
        col = lax.broadcasted_iota(jnp.int32, x.shape, 1)
        s = jnp.where(col < P, x, -x)
        ls = jnp.where(col < CTX, jax.nn.log_sigmoid(s), 0.0)
        o_ref[...] = -jnp.sum(ls, axis=1)

    blk = 512
    return pl.pallas_call(
        body,
        grid=(B // blk,),
        in_specs=[pl.BlockSpec((blk, 128), lambda i: (i, 0))],
        out_specs=pl.BlockSpec((blk,), lambda i: (i,)),
        out_shape=jax.ShapeDtypeStruct((B,), jnp.float32),
    )(dots)


def kernel(input_labels, pos_labels, neg_labels, in_embed, out_embed):
    dots = _sc_dots()(
        input_labels.astype(jnp.int32),
        pos_labels.astype(jnp.int32),
        neg_labels.astype(jnp.int32),
        in_embed,
        out_embed,
    )
    return _tc_loss(dots)


# speedup vs baseline: 2.4041x; 1.3745x over previous
import functools

import jax
import jax.numpy as jnp
from jax import lax
from jax.experimental import pallas as pl
from jax.experimental.pallas import tpu as pltpu
from jax.experimental.pallas import tpu_sc as plsc

VOCAB = 1000000
EMBED = 64
B = 16384
P = 20
N = 100

NC = 2
NS = 16
L = 16
NW = NC * NS
BW = B // NW
BC = 8
NCHUNK = BW // BC
CTX = P + N
NG = 8


def _sc_dots():
    mesh = plsc.VectorSubcoreMesh(core_axis_name="c", subcore_axis_name="s")

    @functools.partial(
        pl.kernel,
        mesh=mesh,
        out_type=jax.ShapeDtypeStruct((B, 128), jnp.float32),
        compiler_params=pltpu.CompilerParams(
            needs_layout_passes=False, use_tc_tiling_on_sc=False),
        scratch_types=[
            pltpu.VMEM((BC,), jnp.int32),
            pltpu.VMEM((BC,), jnp.int32),
            pltpu.VMEM((BC, P), jnp.int32),
            pltpu.VMEM((BC, P), jnp.int32),
            pltpu.VMEM((BC, N), jnp.int32),
            pltpu.VMEM((BC, N), jnp.int32),
            pltpu.VMEM((BC, EMBED), jnp.float32),
            pltpu.VMEM((BC, EMBED), jnp.float32),
            pltpu.VMEM((BC * CTX + 8, EMBED), jnp.float32),
            pltpu.VMEM((BC * CTX + 8, EMBED), jnp.float32),
            pltpu.VMEM((BC, 128), jnp.float32),
            pltpu.VMEM((BC, 128), jnp.float32),
            pltpu.SemaphoreType.DMA,
            pltpu.SemaphoreType.DMA,
            pltpu.SemaphoreType.DMA,
            pltpu.SemaphoreType.DMA,
            pltpu.SemaphoreType.DMA,
            pltpu.SemaphoreType.DMA,
        ],
    )
    def k(input_hbm, pos_hbm, neg_hbm, ine_hbm, oute_hbm, dots_hbm,
          inb0, inb1, pb0, pb1, nb0, nb1, ub0, ub1, rw0, rw1, os0, os1,
          sg0, sg1, so0, so1, si0, si1):
        inb, pb, nb, ub, rw, osb = (inb0, inb1), (pb0, pb1), (nb0, nb1), \
            (ub0, ub1), (rw0, rw1), (os0, os1)
        sg, so, si = (sg0, sg1), (so0, so1), (si0, si1)
        wid = lax.axis_index("s") * NC + lax.axis_index("c")
        lane = lax.iota(jnp.int32, L)
        lane16 = lane * L

        def stage_idx(ci, kb):
            b0 = wid * BW + ci * BC
            pltpu.async_copy(input_hbm.at[pl.ds(b0, BC)], inb[kb], si[kb])
            pltpu.async_copy(pos_hbm.at[pl.ds(b0, BC)], pb[kb], si[kb])
            pltpu.async_copy(neg_hbm.at[pl.ds(b0, BC)], nb[kb], si[kb])

        def wait_idx(kb):
            pltpu.make_async_copy(
                input_hbm.at[pl.ds(0, BC)], inb[kb], si[kb]).wait()
            pltpu.make_async_copy(
                pos_hbm.at[pl.ds(0, BC)], pb[kb], si[kb]).wait()
            pltpu.make_async_copy(
                neg_hbm.at[pl.ds(0, BC)], nb[kb], si[kb]).wait()

        def fire_gathers(kb):
            pltpu.async_copy(ine_hbm.at[inb[kb]], ub[kb], sg[kb])
            for b in range(BC):
                pltpu.async_copy(oute_hbm.at[pb[kb].at[b]],
                                 rw[kb].at[pl.ds(b * CTX, P)], sg[kb])
                pltpu.async_copy(oute_hbm.at[nb[kb].at[b]],
                                 rw[kb].at[pl.ds(b * CTX + P, N)], sg[kb])

        def wait_gathers(kb):
            pltpu.make_async_copy(
                ine_hbm.at[pl.ds(0, BC)], ub[kb], sg[kb]).wait()
            pltpu.make_async_copy(
                oute_hbm.at[pl.ds(0, BC * CTX)],
                rw[kb].at[pl.ds(0, BC * CTX)], sg[kb]).wait()

        def compute(kb):
            rows, ubuf, ostage = rw[kb], ub[kb], osb[kb]

            def b_body(b, _):
                u0 = ubuf[b, pl.ds(0, L)]
                u1 = ubuf[b, pl.ds(L, L)]
                u2 = ubuf[b, pl.ds(2 * L, L)]
                u3 = ubuf[b, pl.ds(3 * L, L)]

                def g_body(g, _):
                    r0 = b * CTX + g * L
                    acc = jnp.zeros((L,), jnp.float32)
                    for c in range(L):
                        r = r0 + c
                        p = ((rows[r, pl.ds(0, L)] * u0
                              + rows[r, pl.ds(L, L)] * u1)
                             + (rows[r, pl.ds(2 * L, L)] * u2
                                + rows[r, pl.ds(3 * L, L)] * u3))
                        acc = jnp.where(lane == c, jnp.sum(p), acc)
                    ostage[b, pl.ds(g * L, L)] = acc
                    return 0

                lax.fori_loop(0, NG, g_body, 0)
                return 0

            lax.fori_loop(0, BC, b_body, 0)

        stage_idx(0, 0)
        stage_idx(1, 1)
        wait_idx(0)
        fire_gathers(0)

        def loop_body(i, _):
            for kb in range(2):
                ci = 2 * i + kb

                @pl.when(ci + 1 < NCHUNK)
                def _():
                    wait_idx(1 - kb)
                    fire_gathers(1 - kb)

                wait_gathers(kb)

                @pl.when(ci + 2 < NCHUNK)
                def _():
                    stage_idx(ci + 2, kb)

                @pl.when(ci >= 2)
                def _():
                    b0_old = wid * BW + (ci - 2) * BC
                    pltpu.make_async_copy(
                        osb[kb], dots_hbm.at[pl.ds(b0_old, BC)],
                        so[kb]).wait()

                compute(kb)
                b0 = wid * BW + ci * BC
                pltpu.async_copy(osb[kb], dots_hbm.at[pl.ds(b0, BC)], so[kb])
            return 0

        lax.fori_loop(0, NCHUNK // 2, loop_body, 0)
        for kb in range(2):
            b0_last = wid * BW + (NCHUNK - 2 + kb) * BC
            pltpu.make_async_copy(
                osb[kb], dots_hbm.at[pl.ds(b0_last, BC)], so[kb]).wait()

    return k


def _tc_loss(dots):
    def body(d_ref, o_ref):
        x = d_ref[...]
        col = lax.broadcasted_iota(jnp.int32, x.shape, 1)
        s = jnp.where(col < P, x, -x)
        ls = jnp.where(col < CTX, jax.nn.log_sigmoid(s), 0.0)
        o_ref[...] = -jnp.sum(ls, axis=1)

    blk = 512
    return pl.pallas_call(
        body,
        grid=(B // blk,),
        in_specs=[pl.BlockSpec((blk, 128), lambda i: (i, 0))],
        out_specs=pl.BlockSpec((blk,), lambda i: (i,)),
        out_shape=jax.ShapeDtypeStruct((B,), jnp.float32),
    )(dots)


def kernel(input_labels, pos_labels, neg_labels, in_embed, out_embed):
    dots = _sc_dots()(
        input_labels.astype(jnp.int32),
        pos_labels.astype(jnp.int32),
        neg_labels.astype(jnp.int32),
        in_embed,
        out_embed,
    )
    return _tc_loss(dots)


# speedup vs baseline: 2.4050x; 1.0003x over previous
import functools

import jax
import jax.numpy as jnp
from jax import lax
from jax.experimental import pallas as pl
from jax.experimental.pallas import tpu as pltpu
from jax.experimental.pallas import tpu_sc as plsc

VOCAB = 1000000
EMBED = 64
B = 16384
P = 20
N = 100

NC = 2
NS = 16
L = 16
NW = NC * NS
BW = B // NW
BC = 8
NCHUNK = BW // BC
CTX = P + N
NG = 8


def _sc_dots():
    mesh = plsc.VectorSubcoreMesh(core_axis_name="c", subcore_axis_name="s")

    @functools.partial(
        pl.kernel,
        mesh=mesh,
        out_type=jax.ShapeDtypeStruct((B, 128), jnp.float32),
        compiler_params=pltpu.CompilerParams(
            needs_layout_passes=False, use_tc_tiling_on_sc=False),
        scratch_types=[
            pltpu.VMEM((BC,), jnp.int32),
            pltpu.VMEM((BC,), jnp.int32),
            pltpu.VMEM((BC, P), jnp.int32),
            pltpu.VMEM((BC, P), jnp.int32),
            pltpu.VMEM((BC, N), jnp.int32),
            pltpu.VMEM((BC, N), jnp.int32),
            pltpu.VMEM((BC, EMBED), jnp.float32),
            pltpu.VMEM((BC, EMBED), jnp.float32),
            pltpu.VMEM((BC * CTX + 8, EMBED), jnp.float32),
            pltpu.VMEM((BC * CTX + 8, EMBED), jnp.float32),
            pltpu.VMEM((BC, 128), jnp.float32),
            pltpu.VMEM((BC, 128), jnp.float32),
            pltpu.SemaphoreType.DMA,
            pltpu.SemaphoreType.DMA,
            pltpu.SemaphoreType.DMA,
            pltpu.SemaphoreType.DMA,
            pltpu.SemaphoreType.DMA,
            pltpu.SemaphoreType.DMA,
        ],
    )
    def k(input_hbm, pos_hbm, neg_hbm, ine_hbm, oute_hbm, dots_hbm,
          inb0, inb1, pb0, pb1, nb0, nb1, ub0, ub1, rw0, rw1, os0, os1,
          sg0, sg1, so0, so1, si0, si1):
        inb, pb, nb, ub, rw, osb = (inb0, inb1), (pb0, pb1), (nb0, nb1), \
            (ub0, ub1), (rw0, rw1), (os0, os1)
        sg, so, si = (sg0, sg1), (so0, so1), (si0, si1)
        wid = lax.axis_index("s") * NC + lax.axis_index("c")
        lane = lax.iota(jnp.int32, L)

        def stage_idx(ci, kb):
            b0 = wid * BW + ci * BC
            pltpu.async_copy(input_hbm.at[pl.ds(b0, BC)], inb[kb], si[kb])
            pltpu.async_copy(pos_hbm.at[pl.ds(b0, BC)], pb[kb], si[kb])
            pltpu.async_copy(neg_hbm.at[pl.ds(b0, BC)], nb[kb], si[kb])

        def wait_idx(kb):
            pltpu.make_async_copy(
                input_hbm.at[pl.ds(0, BC)], inb[kb], si[kb]).wait()
            pltpu.make_async_copy(
                pos_hbm.at[pl.ds(0, BC)], pb[kb], si[kb]).wait()
            pltpu.make_async_copy(
                neg_hbm.at[pl.ds(0, BC)], nb[kb], si[kb]).wait()

        def fire_gathers(kb):
            pltpu.async_copy(ine_hbm.at[inb[kb]], ub[kb], sg[kb])
            for b in range(BC):
                pltpu.async_copy(oute_hbm.at[pb[kb].at[b]],
                                 rw[kb].at[pl.ds(b * CTX, P)], sg[kb])
                pltpu.async_copy(oute_hbm.at[nb[kb].at[b]],
                                 rw[kb].at[pl.ds(b * CTX + P, N)], sg[kb])

        def wait_gathers(kb):
            pltpu.make_async_copy(
                ine_hbm.at[pl.ds(0, BC)], ub[kb], sg[kb]).wait()
            pltpu.make_async_copy(
                oute_hbm.at[pl.ds(0, BC * CTX)],
                rw[kb].at[pl.ds(0, BC * CTX)], sg[kb]).wait()

        def compute(kb):
            rows, ubuf, ostage = rw[kb], ub[kb], osb[kb]

            def b_body(b, _):
                u0 = ubuf[b, pl.ds(0, L)]
                u1 = ubuf[b, pl.ds(L, L)]
                u2 = ubuf[b, pl.ds(2 * L, L)]
                u3 = ubuf[b, pl.ds(3 * L, L)]

                def g_body(g, _):
                    r0 = b * CTX + g * L
                    acc = jnp.zeros((L,), jnp.float32)
                    for c in range(L):
                        r = r0 + c
                        p = ((rows[r, pl.ds(0, L)] * u0
                              + rows[r, pl.ds(L, L)] * u1)
                             + (rows[r, pl.ds(2 * L, L)] * u2
                                + rows[r, pl.ds(3 * L, L)] * u3))
                        acc = jnp.where(lane == c, jnp.sum(p), acc)
                    ostage[b, pl.ds(g * L, L)] = acc
                    return 0

                lax.fori_loop(0, NG, g_body, 0)
                return 0

            lax.fori_loop(0, BC, b_body, 0)

        stage_idx(0, 0)
        stage_idx(1, 1)
        wait_idx(0)
        fire_gathers(0)

        def loop_body(i, _):
            for kb in range(2):
                ci = 2 * i + kb

                @pl.when(ci + 1 < NCHUNK)
                def _():
                    wait_idx(1 - kb)
                    fire_gathers(1 - kb)

                wait_gathers(kb)

                @pl.when(ci + 2 < NCHUNK)
                def _():
                    stage_idx(ci + 2, kb)

                @pl.when(ci >= 2)
                def _():
                    b0_old = wid * BW + (ci - 2) * BC
                    pltpu.make_async_copy(
                        osb[kb], dots_hbm.at[pl.ds(b0_old, BC)],
                        so[kb]).wait()

                compute(kb)
                b0 = wid * BW + ci * BC
                pltpu.async_copy(osb[kb], dots_hbm.at[pl.ds(b0, BC)], so[kb])
            return 0

        lax.fori_loop(0, NCHUNK // 2, loop_body, 0)
        for kb in range(2):
            b0_last = wid * BW + (NCHUNK - 2 + kb) * BC
            pltpu.make_async_copy(
                osb[kb], dots_hbm.at[pl.ds(b0_last, BC)], so[kb]).wait()

    return k


def _tc_loss(dots):
    def body(d_ref, o_ref):
        x = d_ref[...]
        col = lax.broadcasted_iota(jnp.int32, x.shape, 1)
        s = jnp.where(col < P, x, -x)
        ls = jnp.where(col < CTX, jax.nn.log_sigmoid(s), 0.0)
        o_ref[...] = -jnp.sum(ls, axis=1)

    blk = 512
    return pl.pallas_call(
        body,
        grid=(B // blk,),
        in_specs=[pl.BlockSpec((blk, 128), lambda i: (i, 0))],
        out_specs=pl.BlockSpec((blk,), lambda i: (i,)),
        out_shape=jax.ShapeDtypeStruct((B,), jnp.float32),
    )(dots)


def kernel(input_labels, pos_labels, neg_labels, in_embed, out_embed):
    dots = _sc_dots()(
        input_labels.astype(jnp.int32),
        pos_labels.astype(jnp.int32),
        neg_labels.astype(jnp.int32),
        in_embed,
        out_embed,
    )
    return _tc_loss(dots)
